# baseline (speedup 1.0000x reference)
import functools

import jax
import jax.numpy as jnp
from jax import lax
from jax.experimental import pallas as pl
from jax.experimental.pallas import tpu as pltpu
from jax.experimental.pallas import tpu_sc as plsc


def _tc_precompute_body(af_ref, w1_ref, b1_ref, w2_ref, b2_ref, wp_ref, p_ref):
    x = af_ref[...]
    xb = x.astype(jnp.bfloat16)
    t = jnp.dot(xb, w1_ref[...], preferred_element_type=jnp.float32) + b1_ref[...]
    t = t * (1.0 / (1.0 + jnp.exp(-t)))
    w = jnp.dot(t.astype(jnp.bfloat16), w2_ref[...],
                preferred_element_type=jnp.float32) + b2_ref[...]
    s = jax.nn.softmax(w, axis=-1)
    d = x.shape[1]
    h_cnt = w.shape[1]
    y = jnp.dot(xb, wp_ref[...], preferred_element_type=jnp.float32)
    acc = s[:, 0:1] * y[:, :d]
    for h in range(1, h_cnt):
        acc = acc + s[:, h:h + 1] * y[:, h * d:(h + 1) * d]
    p_ref[...] = acc


def _tc_precompute(af, W1, b1, W2, b2, Wp, blk):
    n, d = af.shape
    hid = W1.shape[1]
    h = W2.shape[1]
    grid = n // blk
    return pl.pallas_call(
        _tc_precompute_body,
        grid=(grid,),
        in_specs=[
            pl.BlockSpec((blk, d), lambda i: (i, 0)),
            pl.BlockSpec((d, hid), lambda i: (0, 0)),
            pl.BlockSpec((1, hid), lambda i: (0, 0)),
            pl.BlockSpec((hid, h), lambda i: (0, 0)),
            pl.BlockSpec((1, h), lambda i: (0, 0)),
            pl.BlockSpec((d, h * d), lambda i: (0, 0)),
        ],
        out_specs=pl.BlockSpec((blk, d), lambda i: (i, 0)),
        out_shape=jax.ShapeDtypeStruct((n, d), jnp.float32),
    )(af, W1.astype(jnp.bfloat16), b1.reshape(1, hid),
      W2.astype(jnp.bfloat16), b2.reshape(1, h),
      Wp.reshape(h, d, d).transpose(1, 0, 2).reshape(d, h * d)
      .astype(jnp.bfloat16))


def _sc_pool(p, idx, bp):
    n, d = p.shape
    b, a = idx.shape
    nchunk = d // 16
    info = plsc.get_sparse_core_info()
    nc, ns = info.num_cores, info.num_subcores
    nw = nc * ns
    cb = b // nw
    mesh = plsc.VectorSubcoreMesh(core_axis_name="c", subcore_axis_name="s")

    @functools.partial(
        pl.kernel,
        mesh=mesh,
        out_type=jax.ShapeDtypeStruct((b, d), jnp.float32),
        scratch_types=[
            pltpu.VMEM((cb, a), jnp.int32),
            pltpu.VMEM((4, a, d), jnp.float32),
            pltpu.VMEM((d,), jnp.float32),
            pltpu.VMEM((cb, d), jnp.float32),
            pltpu.SemaphoreType.DMA,
            pltpu.SemaphoreType.DMA,
            pltpu.SemaphoreType.DMA,
            pltpu.SemaphoreType.DMA,
        ],
    )
    def k(p_hbm, idx_hbm, bp_hbm, out_hbm, idx_v, rows_v, bp_v, out_v,
          sem0, sem1, sem2, sem3):
        wid = lax.axis_index("s") * nc + lax.axis_index("c")
        base = wid * cb
        pltpu.sync_copy(idx_hbm.at[pl.ds(base, cb), :], idx_v)
        pltpu.sync_copy(bp_hbm, bp_v)
        sems = (sem0, sem1, sem2, sem3)

        def start(j, buf):
            pltpu.make_async_copy(p_hbm.at[idx_v.at[j]], rows_v.at[buf],
                                  sems[buf]).start()

        def process(j, buf):
            pltpu.make_async_copy(p_hbm.at[idx_v.at[j]], rows_v.at[buf],
                                  sems[buf]).wait()

            def red(ai, acc_in):
                return tuple(acc_in[c] + rows_v[buf, ai, pl.ds(c * 16, 16)]
                             for c in range(nchunk))

            accs = plsc.parallel_loop(
                0, a, unroll=8,
                carry=tuple(jnp.zeros((16,), jnp.float32)
                            for _ in range(nchunk)))(red)
            for c in range(nchunk):
                v = accs[c] + bp_v[pl.ds(c * 16, 16)]
                out_v[j, pl.ds(c * 16, 16)] = v * (1.0 / (1.0 + jnp.exp(-v)))

        nbuf = 4
        for i in range(nbuf):
            start(i, i)

        def body(jj, carry):
            for buf in range(nbuf):
                j = jj * nbuf + buf
                process(j, buf)

                @pl.when(j + nbuf < cb)
                def _():
                    start(j + nbuf, buf)
            return carry

        lax.fori_loop(0, cb // nbuf, body, 0)
        pltpu.sync_copy(out_v, out_hbm.at[pl.ds(base, cb), :])

    return k(p, idx, bp)


def kernel(atom_fea, crystal_atom_idx, W1, b1, W2, b2, Wp, bp):
    n = atom_fea.shape[0]
    blk = 5000 if n % 5000 == 0 else 8
    p = _tc_precompute(atom_fea, W1, b1, W2, b2, Wp, blk)
    return _sc_pool(p, crystal_atom_idx.astype(jnp.int32), bp)


# speedup vs baseline: 1.5070x; 1.5070x over previous
import functools

import jax
import jax.numpy as jnp
from jax import lax
from jax.experimental import pallas as pl
from jax.experimental.pallas import tpu as pltpu
from jax.experimental.pallas import tpu_sc as plsc


def _tc_precompute_body(af_ref, w1_ref, b1_ref, w2_ref, b2_ref, wp_ref, p_ref):
    x = af_ref[...]
    xb = x.astype(jnp.bfloat16)
    t = jnp.dot(xb, w1_ref[...], preferred_element_type=jnp.float32) + b1_ref[...]
    t = t * (1.0 / (1.0 + jnp.exp(-t)))
    w = jnp.dot(t.astype(jnp.bfloat16), w2_ref[...],
                preferred_element_type=jnp.float32) + b2_ref[...]
    s = jax.nn.softmax(w, axis=-1)
    d = x.shape[1]
    h_cnt = w.shape[1]
    y = jnp.dot(xb, wp_ref[...], preferred_element_type=jnp.float32)
    acc = s[:, 0:1] * y[:, :d]
    for h in range(1, h_cnt):
        acc = acc + s[:, h:h + 1] * y[:, h * d:(h + 1) * d]
    p_ref[...] = acc


def _tc_precompute(af, W1, b1, W2, b2, Wp, blk):
    n, d = af.shape
    hid = W1.shape[1]
    h = W2.shape[1]
    grid = n // blk
    return pl.pallas_call(
        _tc_precompute_body,
        grid=(grid,),
        in_specs=[
            pl.BlockSpec((blk, d), lambda i: (i, 0)),
            pl.BlockSpec((d, hid), lambda i: (0, 0)),
            pl.BlockSpec((1, hid), lambda i: (0, 0)),
            pl.BlockSpec((hid, h), lambda i: (0, 0)),
            pl.BlockSpec((1, h), lambda i: (0, 0)),
            pl.BlockSpec((d, h * d), lambda i: (0, 0)),
        ],
        out_specs=pl.BlockSpec((blk, d), lambda i: (i, 0)),
        out_shape=jax.ShapeDtypeStruct((n, d), jnp.float32),
    )(af, W1.astype(jnp.bfloat16), b1.reshape(1, hid),
      W2.astype(jnp.bfloat16), b2.reshape(1, h),
      Wp.reshape(h, d, d).transpose(1, 0, 2).reshape(d, h * d)
      .astype(jnp.bfloat16))


def _sc_pool(p, idx, bp):
    n, d = p.shape
    b, a = idx.shape
    nchunk = d // 16
    info = plsc.get_sparse_core_info()
    nc, ns = info.num_cores, info.num_subcores
    nw = nc * ns
    cb = b // nw
    mesh = plsc.VectorSubcoreMesh(core_axis_name="c", subcore_axis_name="s")

    @functools.partial(
        pl.kernel,
        mesh=mesh,
        out_type=jax.ShapeDtypeStruct((b, d), jnp.float32),
        scratch_types=[
            pltpu.VMEM((cb, a), jnp.int32),
            pltpu.VMEM((4, a, d), jnp.float32),
            pltpu.VMEM((d,), jnp.float32),
            pltpu.VMEM((cb, d), jnp.float32),
            pltpu.SemaphoreType.DMA,
            pltpu.SemaphoreType.DMA,
            pltpu.SemaphoreType.DMA,
            pltpu.SemaphoreType.DMA,
        ],
    )
    def k(p_hbm, idx_hbm, bp_hbm, out_hbm, idx_v, rows_v, bp_v, out_v,
          sem0, sem1, sem2, sem3):
        wid = lax.axis_index("s") * nc + lax.axis_index("c")
        base = wid * cb
        pltpu.sync_copy(idx_hbm.at[pl.ds(base, cb), :], idx_v)
        pltpu.sync_copy(bp_hbm, bp_v)
        sems = (sem0, sem1, sem2, sem3)

        def start(j, buf):
            pltpu.make_async_copy(p_hbm.at[idx_v.at[j]], rows_v.at[buf],
                                  sems[buf]).start()

        def process(j, buf):
            pltpu.make_async_copy(p_hbm.at[idx_v.at[j]], rows_v.at[buf],
                                  sems[buf]).wait()

            def red(ai, acc_in):
                return tuple(acc_in[c] + rows_v[buf, ai, pl.ds(c * 16, 16)]
                             for c in range(nchunk))

            accs = plsc.parallel_loop(
                0, a, unroll=8,
                carry=tuple(jnp.zeros((16,), jnp.float32)
                            for _ in range(nchunk)))(red)
            for c in range(nchunk):
                v = accs[c] + bp_v[pl.ds(c * 16, 16)]
                out_v[j, pl.ds(c * 16, 16)] = v * (1.0 / (1.0 + jnp.exp(-v)))

        nbuf = 4
        for i in range(nbuf):
            start(i, i)

        def body(jj, carry):
            for buf in range(nbuf):
                j = jj * nbuf + buf
                process(j, buf)

                @pl.when(j + nbuf < cb)
                def _():
                    start(j + nbuf, buf)
            return carry

        lax.fori_loop(0, cb // nbuf, body, 0)
        pltpu.sync_copy(out_v, out_hbm.at[pl.ds(base, cb), :])

    return k(p, idx, bp)


def kernel(atom_fea, crystal_atom_idx, W1, b1, W2, b2, Wp, bp):
    n = atom_fea.shape[0]
    blk = 4000 if n % 4000 == 0 else 8
    p = _tc_precompute(atom_fea, W1, b1, W2, b2, Wp, blk)
    return _sc_pool(p, crystal_atom_idx.astype(jnp.int32), bp)
